# speedup vs baseline: 5.9717x; 5.9717x over previous
import jax
import jax.numpy as jnp
from jax.experimental import pallas as pl

_B = 256
_BLK = 2000


def _k1_logits_segmax(x_ref, w_ref, b_ref, batch_ref, logit_ref, segmax_ref):
    i = pl.program_id(0)
    lg = jnp.dot(x_ref[...], w_ref[...],
                 preferred_element_type=jnp.float32) + b_ref[0, 0]
    logit_ref[...] = lg
    seg = jax.lax.broadcasted_iota(jnp.int32, (1, _B), 1)
    m = batch_ref[...] == seg
    cur = jnp.max(jnp.where(m, lg, -1e30), axis=0, keepdims=True)

    @pl.when(i == 0)
    def _():
        segmax_ref[...] = cur

    @pl.when(i > 0)
    def _():
        segmax_ref[...] = jnp.maximum(segmax_ref[...], cur)


def _k2_exp_denom(l_ref, batch_ref, segmax_ref, ex_ref, den_ref):
    i = pl.program_id(0)
    seg = jax.lax.broadcasted_iota(jnp.int32, (1, _B), 1)
    m = batch_ref[...] == seg
    gmax = jnp.max(jnp.where(m, segmax_ref[...], -1e30),
                   axis=1, keepdims=True)
    ex = jnp.exp(l_ref[...] - gmax)
    ex_ref[...] = ex
    cur = jnp.sum(jnp.where(m, ex, 0.0), axis=0, keepdims=True)

    @pl.when(i == 0)
    def _():
        den_ref[...] = cur

    @pl.when(i > 0)
    def _():
        den_ref[...] = den_ref[...] + cur


def _k3_pool(x_ref, batch_ref, ex_ref, den_ref, out_ref):
    i = pl.program_id(0)
    seg = jax.lax.broadcasted_iota(jnp.int32, (1, _B), 1)
    m = batch_ref[...] == seg
    ginv = jnp.sum(jnp.where(m, 1.0 / den_ref[...], 0.0),
                   axis=1, keepdims=True)
    y = x_ref[...] * (ex_ref[...] * ginv)
    part = jax.lax.dot_general(m.astype(jnp.float32), y,
                               (((0,), (0,)), ((), ())),
                               preferred_element_type=jnp.float32)

    @pl.when(i == 0)
    def _():
        out_ref[...] = part

    @pl.when(i > 0)
    def _():
        out_ref[...] = out_ref[...] + part


def kernel(x, batch, w_x, bias):
    n, d = x.shape
    grid = (n // _BLK,)
    batch2 = batch.astype(jnp.int32).reshape(n, 1)
    w2 = w_x.reshape(d, 1)
    b2 = bias.reshape(1, 1)

    row_spec = pl.BlockSpec((_BLK, 1), lambda i: (i, 0))
    full_x_spec = pl.BlockSpec((_BLK, d), lambda i: (i, 0))
    seg_spec = pl.BlockSpec((1, _B), lambda i: (0, 0))

    logits, segmax = pl.pallas_call(
        _k1_logits_segmax,
        grid=grid,
        in_specs=[full_x_spec,
                  pl.BlockSpec((d, 1), lambda i: (0, 0)),
                  pl.BlockSpec((1, 1), lambda i: (0, 0)),
                  row_spec],
        out_specs=(row_spec, seg_spec),
        out_shape=(jax.ShapeDtypeStruct((n, 1), jnp.float32),
                   jax.ShapeDtypeStruct((1, _B), jnp.float32)),
    )(x, w2, b2, batch2)

    ex, denom = pl.pallas_call(
        _k2_exp_denom,
        grid=grid,
        in_specs=[row_spec, row_spec, seg_spec],
        out_specs=(row_spec, seg_spec),
        out_shape=(jax.ShapeDtypeStruct((n, 1), jnp.float32),
                   jax.ShapeDtypeStruct((1, _B), jnp.float32)),
    )(logits, batch2, segmax)

    out = pl.pallas_call(
        _k3_pool,
        grid=grid,
        in_specs=[full_x_spec, row_spec, row_spec, seg_spec],
        out_specs=pl.BlockSpec((_B, d), lambda i: (0, 0)),
        out_shape=jax.ShapeDtypeStruct((_B, d), jnp.float32),
    )(x, batch2, ex, denom)

    return out


# speedup vs baseline: 6.3424x; 1.0621x over previous
import jax
import jax.numpy as jnp
from jax import lax
from jax.experimental import pallas as pl
from jax.experimental.pallas import tpu as pltpu
from jax.experimental.pallas import tpu_sc as plsc

_B = 256
_D = 128
_N = 100000
_BLK = 2000

_ST_CNT = 6272
_ST_LAST = _N - 15 * _ST_CNT
_PL_CNT = 3136
_PL_LAST = _N - 31 * _PL_CNT
_XCH = 224
_XG = _XCH // 16
_PL_NCH = _PL_CNT // _XCH
_PL_NCH_LAST = 12
_PL_TAIL = _PL_LAST - _PL_NCH_LAST * _XCH


def _k1_logits(x_ref, w_ref, b_ref, logit_ref):
    logit_ref[...] = jnp.dot(x_ref[...], w_ref[...],
                             preferred_element_type=jnp.float32) + b_ref[0, 0]


def _k2_sum(p_ref, out_ref):
    out_ref[...] = jnp.sum(p_ref[...], axis=0)


def _sc_body(x_hbm, batch_hbm, logit_hbm, part_hbm,
             bbuf, lbuf, abuf, table, cbuf, part, gmax, invd, acc, xbuf, shsc):
    c = lax.axis_index("c")
    s = lax.axis_index("s")
    w = c * 16 + s
    lanes = lax.broadcasted_iota(jnp.int32, (16,), 0)

    o1 = s * _ST_CNT
    nr1 = jnp.where(s == 15, _ST_LAST // 16, _ST_CNT // 16)

    @pl.when(s < 15)
    def _():
        pltpu.sync_copy(batch_hbm.at[pl.ds(o1, _ST_CNT)], bbuf)
        pltpu.sync_copy(logit_hbm.at[pl.ds(o1, _ST_CNT)], lbuf)

    @pl.when(s == 15)
    def _():
        pltpu.sync_copy(batch_hbm.at[pl.ds(o1, _ST_LAST)],
                        bbuf.at[pl.ds(0, _ST_LAST)])
        pltpu.sync_copy(logit_hbm.at[pl.ds(o1, _ST_LAST)],
                        lbuf.at[pl.ds(0, _ST_LAST)])

    def init_t(j, _):
        table[j, :] = jnp.full((16,), -1e30, jnp.float32)
        return 0
    lax.fori_loop(0, _B, init_t, 0)

    def maxloop(i, _):
        b = bbuf[pl.ds(i * 16, 16)]
        l = lbuf[pl.ds(i * 16, 16)]
        cur = plsc.load_gather(table, [b, lanes])
        plsc.store_scatter(table, [b, lanes], jnp.maximum(cur, l))
        return 0
    lax.fori_loop(0, nr1, maxloop, 0)

    def red_max(g, _):
        rows = g * 16 + lanes
        m = jnp.full((16,), -1e30, jnp.float32)
        for col in range(16):
            cv = plsc.load_gather(table, [rows, jnp.full((16,), col, jnp.int32)])
            m = jnp.maximum(m, cv)
        part[pl.ds(g * 16, 16)] = m
        return 0
    lax.fori_loop(0, _B // 16, red_max, 0)

    pltpu.sync_copy(part, shsc.at[s])
    plsc.subcore_barrier()
    pltpu.sync_copy(shsc, cbuf)
    plsc.subcore_barrier()

    def gmax_loop(g, _):
        m = cbuf[0, pl.ds(g * 16, 16)]
        for t in range(1, 16):
            m = jnp.maximum(m, cbuf[t, pl.ds(g * 16, 16)])
        gmax[pl.ds(g * 16, 16)] = m
        return 0
    lax.fori_loop(0, _B // 16, gmax_loop, 0)

    def zero_t(j, _):
        table[j, :] = jnp.zeros((16,), jnp.float32)
        return 0
    lax.fori_loop(0, _B, zero_t, 0)

    def exploop(i, _):
        b = bbuf[pl.ds(i * 16, 16)]
        l = lbuf[pl.ds(i * 16, 16)]
        ex = jnp.exp(l - plsc.load_gather(gmax, [b]))
        plsc.addupdate_scatter(table, [b, lanes], ex)
        return 0
    lax.fori_loop(0, nr1, exploop, 0)

    def red_sum(g, _):
        rows = g * 16 + lanes
        m = jnp.zeros((16,), jnp.float32)
        for col in range(16):
            m = m + plsc.load_gather(table, [rows, jnp.full((16,), col, jnp.int32)])
        part[pl.ds(g * 16, 16)] = m
        return 0
    lax.fori_loop(0, _B // 16, red_sum, 0)

    pltpu.sync_copy(part, shsc.at[s])
    plsc.subcore_barrier()
    pltpu.sync_copy(shsc, cbuf)
    plsc.subcore_barrier()

    def invd_loop(g, _):
        m = cbuf[0, pl.ds(g * 16, 16)]
        for t in range(1, 16):
            m = m + cbuf[t, pl.ds(g * 16, 16)]
        invd[pl.ds(g * 16, 16)] = 1.0 / m
        return 0
    lax.fori_loop(0, _B // 16, invd_loop, 0)

    o3 = w * _PL_CNT
    nr3 = jnp.where(w == 31, _PL_LAST // 16, _PL_CNT // 16)

    @pl.when(w < 31)
    def _():
        pltpu.sync_copy(batch_hbm.at[pl.ds(o3, _PL_CNT)],
                        bbuf.at[pl.ds(0, _PL_CNT)])
        pltpu.sync_copy(logit_hbm.at[pl.ds(o3, _PL_CNT)],
                        lbuf.at[pl.ds(0, _PL_CNT)])

    @pl.when(w == 31)
    def _():
        pltpu.sync_copy(batch_hbm.at[pl.ds(o3, _PL_LAST)],
                        bbuf.at[pl.ds(0, _PL_LAST)])
        pltpu.sync_copy(logit_hbm.at[pl.ds(o3, _PL_LAST)],
                        lbuf.at[pl.ds(0, _PL_LAST)])

    def attn_loop(i, _):
        b = bbuf[pl.ds(i * 16, 16)]
        l = lbuf[pl.ds(i * 16, 16)]
        a = jnp.exp(l - plsc.load_gather(gmax, [b])) * plsc.load_gather(invd, [b])
        abuf[pl.ds(i * 16, 16)] = a
        return 0
    lax.fori_loop(0, nr3, attn_loop, 0)

    def zero_acc(j, _):
        for k in range(8):
            acc[j, pl.ds(k * 16, 16)] = jnp.zeros((16,), jnp.float32)
        return 0
    lax.fori_loop(0, _B, zero_acc, 0)

    def group_body(i, g):
        b16 = bbuf[pl.ds(i * 16, 16)]
        a16 = abuf[pl.ds(i * 16, 16)]
        for j in range(16):
            b = b16[j]
            a = a16[j]
            for k in range(8):
                plsc.addupdate(acc.at[b, pl.ds(k * 16, 16)],
                               a * xbuf[g * 16 + j, pl.ds(k * 16, 16)])

    xbase = o3

    def chunk_loop(ci, _):
        pltpu.sync_copy(x_hbm.at[pl.ds(xbase + ci * _XCH, _XCH), :], xbuf)

        def gb(g, _):
            group_body(ci * _XG + g, g)
            return 0
        lax.fori_loop(0, _XG, gb, 0)
        return 0
    nfull = jnp.where(w == 31, _PL_NCH_LAST, _PL_NCH)
    lax.fori_loop(0, nfull, chunk_loop, 0)

    @pl.when(w == 31)
    def _():
        pltpu.sync_copy(x_hbm.at[pl.ds(xbase + _PL_NCH_LAST * _XCH, _PL_TAIL), :],
                        xbuf.at[pl.ds(0, _PL_TAIL), :])

        def gb(g, _):
            group_body(_PL_NCH_LAST * _XG + g, g)
            return 0
        lax.fori_loop(0, _PL_TAIL // 16, gb, 0)

    pltpu.sync_copy(acc, part_hbm.at[w])


_sc_pool = pl.kernel(
    _sc_body,
    out_type=jax.ShapeDtypeStruct((32, _B, _D), jnp.float32),
    mesh=plsc.VectorSubcoreMesh(core_axis_name="c", subcore_axis_name="s",
                                num_cores=2, num_subcores=16),
    compiler_params=pltpu.CompilerParams(needs_layout_passes=False),
    scratch_types=[
        pltpu.VMEM((_ST_CNT,), jnp.int32),
        pltpu.VMEM((_ST_CNT,), jnp.float32),
        pltpu.VMEM((_PL_CNT,), jnp.float32),
        pltpu.VMEM((_B, 16), jnp.float32),
        pltpu.VMEM((16, _B), jnp.float32),
        pltpu.VMEM((_B,), jnp.float32),
        pltpu.VMEM((_B,), jnp.float32),
        pltpu.VMEM((_B,), jnp.float32),
        pltpu.VMEM((_B, _D), jnp.float32),
        pltpu.VMEM((_XCH, _D), jnp.float32),
        pltpu.VMEM_SHARED((16, _B), jnp.float32),
    ],
)


def kernel(x, batch, w_x, bias):
    n, d = x.shape
    batch2 = batch.astype(jnp.int32)
    w2 = w_x.reshape(d, 1)
    b2 = bias.reshape(1, 1)

    logits = pl.pallas_call(
        _k1_logits,
        grid=(n // _BLK,),
        in_specs=[pl.BlockSpec((_BLK, d), lambda i: (i, 0)),
                  pl.BlockSpec((d, 1), lambda i: (0, 0)),
                  pl.BlockSpec((1, 1), lambda i: (0, 0))],
        out_specs=pl.BlockSpec((_BLK, 1), lambda i: (i, 0)),
        out_shape=jax.ShapeDtypeStruct((n, 1), jnp.float32),
    )(x, w2, b2)

    partials = _sc_pool(x, batch2, logits.reshape(n))

    out = pl.pallas_call(
        _k2_sum,
        in_specs=[pl.BlockSpec((32, _B, d), lambda: (0, 0, 0))],
        out_specs=pl.BlockSpec((_B, d), lambda: (0, 0)),
        out_shape=jax.ShapeDtypeStruct((_B, d), jnp.float32),
    )(partials)

    return out


# speedup vs baseline: 10.4720x; 1.6511x over previous
import jax
import jax.numpy as jnp
from jax import lax
from jax.experimental import pallas as pl
from jax.experimental.pallas import tpu as pltpu
from jax.experimental.pallas import tpu_sc as plsc

_B = 256
_D = 128
_N = 100000
_BLK = 2000

_ST_CNT = 6272
_ST_LAST = _N - 15 * _ST_CNT
_PL_CNT = 3136
_PL_LAST = _N - 31 * _PL_CNT
_XCH = 112
_XG = _XCH // 16
_PL_NCH = _PL_CNT // _XCH
_PL_NCH_LAST = 24
_PL_TAIL = _PL_LAST - _PL_NCH_LAST * _XCH


def _k1_logits(x_ref, w_ref, b_ref, logit_ref):
    logit_ref[...] = jnp.dot(x_ref[...], w_ref[...],
                             preferred_element_type=jnp.float32) + b_ref[0, 0]


def _k2_sum(p_ref, out_ref):
    out_ref[...] = jnp.sum(p_ref[...], axis=0)


def _sc_body(x_hbm, batch_hbm, logit_hbm, part_hbm,
             bbuf, lbuf, abuf, table, cbuf, part, gmax, invd, acc,
             xbuf, xbuf2, xsem, shsc):
    c = lax.axis_index("c")
    s = lax.axis_index("s")
    w = c * 16 + s
    lanes = lax.broadcasted_iota(jnp.int32, (16,), 0)

    o1 = s * _ST_CNT
    nr1 = jnp.where(s == 15, _ST_LAST // 16, _ST_CNT // 16)

    @pl.when(s < 15)
    def _():
        pltpu.sync_copy(batch_hbm.at[pl.ds(o1, _ST_CNT)], bbuf)
        pltpu.sync_copy(logit_hbm.at[pl.ds(o1, _ST_CNT)], lbuf)

    @pl.when(s == 15)
    def _():
        pltpu.sync_copy(batch_hbm.at[pl.ds(o1, _ST_LAST)],
                        bbuf.at[pl.ds(0, _ST_LAST)])
        pltpu.sync_copy(logit_hbm.at[pl.ds(o1, _ST_LAST)],
                        lbuf.at[pl.ds(0, _ST_LAST)])

    def init_t(j, _):
        table[j, :] = jnp.full((16,), -1e30, jnp.float32)
        return 0
    lax.fori_loop(0, _B, init_t, 0)

    def maxloop(i, _):
        b = bbuf[pl.ds(i * 16, 16)]
        l = lbuf[pl.ds(i * 16, 16)]
        cur = plsc.load_gather(table, [b, lanes])
        plsc.store_scatter(table, [b, lanes], jnp.maximum(cur, l))
        return 0
    lax.fori_loop(0, nr1, maxloop, 0)

    def red_max(g, _):
        rows = g * 16 + lanes
        m = jnp.full((16,), -1e30, jnp.float32)
        for col in range(16):
            cv = plsc.load_gather(table, [rows, jnp.full((16,), col, jnp.int32)])
            m = jnp.maximum(m, cv)
        part[pl.ds(g * 16, 16)] = m
        return 0
    lax.fori_loop(0, _B // 16, red_max, 0)

    pltpu.sync_copy(part, shsc.at[s])
    plsc.subcore_barrier()
    pltpu.sync_copy(shsc, cbuf)
    plsc.subcore_barrier()

    def gmax_loop(g, _):
        m = cbuf[0, pl.ds(g * 16, 16)]
        for t in range(1, 16):
            m = jnp.maximum(m, cbuf[t, pl.ds(g * 16, 16)])
        gmax[pl.ds(g * 16, 16)] = m
        return 0
    lax.fori_loop(0, _B // 16, gmax_loop, 0)

    def zero_t(j, _):
        table[j, :] = jnp.zeros((16,), jnp.float32)
        return 0
    lax.fori_loop(0, _B, zero_t, 0)

    def exploop(i, _):
        b = bbuf[pl.ds(i * 16, 16)]
        l = lbuf[pl.ds(i * 16, 16)]
        ex = jnp.exp(l - plsc.load_gather(gmax, [b]))
        plsc.addupdate_scatter(table, [b, lanes], ex)
        return 0
    lax.fori_loop(0, nr1, exploop, 0)

    def red_sum(g, _):
        rows = g * 16 + lanes
        m = jnp.zeros((16,), jnp.float32)
        for col in range(16):
            m = m + plsc.load_gather(table, [rows, jnp.full((16,), col, jnp.int32)])
        part[pl.ds(g * 16, 16)] = m
        return 0
    lax.fori_loop(0, _B // 16, red_sum, 0)

    pltpu.sync_copy(part, shsc.at[s])
    plsc.subcore_barrier()
    pltpu.sync_copy(shsc, cbuf)
    plsc.subcore_barrier()

    def invd_loop(g, _):
        m = cbuf[0, pl.ds(g * 16, 16)]
        for t in range(1, 16):
            m = m + cbuf[t, pl.ds(g * 16, 16)]
        invd[pl.ds(g * 16, 16)] = 1.0 / m
        return 0
    lax.fori_loop(0, _B // 16, invd_loop, 0)

    o3 = w * _PL_CNT
    nr3 = jnp.where(w == 31, _PL_LAST // 16, _PL_CNT // 16)

    @pl.when(w < 31)
    def _():
        pltpu.sync_copy(batch_hbm.at[pl.ds(o3, _PL_CNT)],
                        bbuf.at[pl.ds(0, _PL_CNT)])
        pltpu.sync_copy(logit_hbm.at[pl.ds(o3, _PL_CNT)],
                        lbuf.at[pl.ds(0, _PL_CNT)])

    @pl.when(w == 31)
    def _():
        pltpu.sync_copy(batch_hbm.at[pl.ds(o3, _PL_LAST)],
                        bbuf.at[pl.ds(0, _PL_LAST)])
        pltpu.sync_copy(logit_hbm.at[pl.ds(o3, _PL_LAST)],
                        lbuf.at[pl.ds(0, _PL_LAST)])

    def attn_loop(i, _):
        b = bbuf[pl.ds(i * 16, 16)]
        l = lbuf[pl.ds(i * 16, 16)]
        a = jnp.exp(l - plsc.load_gather(gmax, [b])) * plsc.load_gather(invd, [b])
        abuf[pl.ds(i * 16, 16)] = a
        return 0
    lax.fori_loop(0, nr3, attn_loop, 0)

    def zero_acc(j, _):
        for k in range(8):
            acc[j, pl.ds(k * 16, 16)] = jnp.zeros((16,), jnp.float32)
        return 0
    lax.fori_loop(0, _B, zero_acc, 0)

    def group_body(xb, i, g):
        b16 = bbuf[pl.ds(i * 16, 16)]
        a16 = abuf[pl.ds(i * 16, 16)]
        b_first = b16[0]
        b_last = b16[15]

        @pl.when(b_first == b_last)
        def _():
            regs = [jnp.zeros((16,), jnp.float32) for _ in range(8)]
            for j in range(16):
                a = a16[j]
                for k in range(8):
                    regs[k] = regs[k] + a * xb[g * 16 + j, pl.ds(k * 16, 16)]
            for k in range(8):
                plsc.addupdate(acc.at[b_first, pl.ds(k * 16, 16)], regs[k])

        @pl.when(b_first != b_last)
        def _():
            for j in range(16):
                b = b16[j]
                a = a16[j]
                for k in range(8):
                    plsc.addupdate(acc.at[b, pl.ds(k * 16, 16)],
                                   a * xb[g * 16 + j, pl.ds(k * 16, 16)])

    xbase = o3

    def start_chunk(ci, buf):
        pltpu.async_copy(x_hbm.at[pl.ds(xbase + ci * _XCH, _XCH), :], buf, xsem)

    def wait_chunk(ci, buf):
        pltpu.make_async_copy(x_hbm.at[pl.ds(xbase + ci * _XCH, _XCH), :],
                              buf, xsem).wait()

    def proc_chunk(ci, buf):
        def gb(g, _):
            group_body(buf, ci * _XG + g, g)
            return 0
        lax.fori_loop(0, _XG, gb, 0)

    nfull = jnp.where(w == 31, _PL_NCH_LAST, _PL_NCH)
    start_chunk(0, xbuf)

    def chunk_loop(ci, _):
        @pl.when(ci % 2 == 0)
        def _():
            wait_chunk(ci, xbuf)

            @pl.when(ci + 1 < nfull)
            def _():
                start_chunk(ci + 1, xbuf2)
            proc_chunk(ci, xbuf)

        @pl.when(ci % 2 == 1)
        def _():
            wait_chunk(ci, xbuf2)

            @pl.when(ci + 1 < nfull)
            def _():
                start_chunk(ci + 1, xbuf)
            proc_chunk(ci, xbuf2)
        return 0
    lax.fori_loop(0, nfull, chunk_loop, 0)

    @pl.when(w == 31)
    def _():
        pltpu.sync_copy(x_hbm.at[pl.ds(xbase + _PL_NCH_LAST * _XCH, _PL_TAIL), :],
                        xbuf.at[pl.ds(0, _PL_TAIL), :])

        def gb(g, _):
            group_body(xbuf, _PL_NCH_LAST * _XG + g, g)
            return 0
        lax.fori_loop(0, _PL_TAIL // 16, gb, 0)

    pltpu.sync_copy(acc, part_hbm.at[w])


_sc_pool = pl.kernel(
    _sc_body,
    out_type=jax.ShapeDtypeStruct((32, _B, _D), jnp.float32),
    mesh=plsc.VectorSubcoreMesh(core_axis_name="c", subcore_axis_name="s",
                                num_cores=2, num_subcores=16),
    compiler_params=pltpu.CompilerParams(needs_layout_passes=False),
    scratch_types=[
        pltpu.VMEM((_ST_CNT,), jnp.int32),
        pltpu.VMEM((_ST_CNT,), jnp.float32),
        pltpu.VMEM((_PL_CNT,), jnp.float32),
        pltpu.VMEM((_B, 16), jnp.float32),
        pltpu.VMEM((16, _B), jnp.float32),
        pltpu.VMEM((_B,), jnp.float32),
        pltpu.VMEM((_B,), jnp.float32),
        pltpu.VMEM((_B,), jnp.float32),
        pltpu.VMEM((_B, _D), jnp.float32),
        pltpu.VMEM((_XCH, _D), jnp.float32),
        pltpu.VMEM((_XCH, _D), jnp.float32),
        pltpu.SemaphoreType.DMA,
        pltpu.VMEM_SHARED((16, _B), jnp.float32),
    ],
)


def kernel(x, batch, w_x, bias):
    n, d = x.shape
    batch2 = batch.astype(jnp.int32)
    w2 = w_x.reshape(d, 1)
    b2 = bias.reshape(1, 1)

    logits = pl.pallas_call(
        _k1_logits,
        grid=(n // _BLK,),
        in_specs=[pl.BlockSpec((_BLK, d), lambda i: (i, 0)),
                  pl.BlockSpec((d, 1), lambda i: (0, 0)),
                  pl.BlockSpec((1, 1), lambda i: (0, 0))],
        out_specs=pl.BlockSpec((_BLK, 1), lambda i: (i, 0)),
        out_shape=jax.ShapeDtypeStruct((n, 1), jnp.float32),
    )(x, w2, b2)

    partials = _sc_pool(x, batch2, logits.reshape(n))

    out = pl.pallas_call(
        _k2_sum,
        in_specs=[pl.BlockSpec((32, _B, d), lambda: (0, 0, 0))],
        out_specs=pl.BlockSpec((_B, d), lambda: (0, 0)),
        out_shape=jax.ShapeDtypeStruct((_B, d), jnp.float32),
    )(partials)

    return out


# speedup vs baseline: 12.1911x; 1.1642x over previous
import jax
import jax.numpy as jnp
from jax import lax
from jax.experimental import pallas as pl
from jax.experimental.pallas import tpu as pltpu
from jax.experimental.pallas import tpu_sc as plsc

_B = 256
_D = 128
_N = 100000
_BLK = 10000

_ST_CNT = 6272
_ST_LAST = _N - 15 * _ST_CNT
_PL_CNT = 3136
_PL_LAST = _N - 31 * _PL_CNT
_XCH = 112
_XG = _XCH // 16
_PL_NCH = _PL_CNT // _XCH
_PL_NCH_LAST = 24
_PL_TAIL = _PL_LAST - _PL_NCH_LAST * _XCH


def _k1_logits(x_ref, w_ref, b_ref, logit_ref):
    logit_ref[...] = jnp.dot(x_ref[...], w_ref[...],
                             preferred_element_type=jnp.float32) + b_ref[0, 0]


def _k2_sum(p_ref, out_ref):
    out_ref[...] = jnp.sum(p_ref[...], axis=0)


def _sc_body(x_hbm, batch_hbm, logit_hbm, part_hbm,
             bbuf, lbuf, abuf, table, cbuf, part, gmax, invd, acc,
             xbuf, xbuf2, xsem, shsc):
    c = lax.axis_index("c")
    s = lax.axis_index("s")
    w = c * 16 + s
    lanes = lax.broadcasted_iota(jnp.int32, (16,), 0)

    o1 = s * _ST_CNT
    nr1 = jnp.where(s == 15, _ST_LAST // 16, _ST_CNT // 16)

    @pl.when(s < 15)
    def _():
        pltpu.sync_copy(batch_hbm.at[pl.ds(o1, _ST_CNT)], bbuf)
        pltpu.sync_copy(logit_hbm.at[pl.ds(o1, _ST_CNT)], lbuf)

    @pl.when(s == 15)
    def _():
        pltpu.sync_copy(batch_hbm.at[pl.ds(o1, _ST_LAST)],
                        bbuf.at[pl.ds(0, _ST_LAST)])
        pltpu.sync_copy(logit_hbm.at[pl.ds(o1, _ST_LAST)],
                        lbuf.at[pl.ds(0, _ST_LAST)])

    def init_t(j, _):
        table[j, :] = jnp.full((16,), -1e30, jnp.float32)
        return 0
    lax.fori_loop(0, _B, init_t, 0)

    def maxloop(i, _):
        b = bbuf[pl.ds(i * 16, 16)]
        l = lbuf[pl.ds(i * 16, 16)]
        cur = plsc.load_gather(table, [b, lanes])
        plsc.store_scatter(table, [b, lanes], jnp.maximum(cur, l))
        return 0
    lax.fori_loop(0, nr1, maxloop, 0)

    def red_max(g, _):
        rows = g * 16 + lanes
        m = jnp.full((16,), -1e30, jnp.float32)
        for col in range(16):
            cv = plsc.load_gather(table, [rows, jnp.full((16,), col, jnp.int32)])
            m = jnp.maximum(m, cv)
        part[pl.ds(g * 16, 16)] = m
        return 0
    lax.fori_loop(0, _B // 16, red_max, 0)

    pltpu.sync_copy(part, shsc.at[s])
    plsc.subcore_barrier()
    pltpu.sync_copy(shsc, cbuf)
    plsc.subcore_barrier()

    def gmax_loop(g, _):
        m = cbuf[0, pl.ds(g * 16, 16)]
        for t in range(1, 16):
            m = jnp.maximum(m, cbuf[t, pl.ds(g * 16, 16)])
        gmax[pl.ds(g * 16, 16)] = m
        return 0
    lax.fori_loop(0, _B // 16, gmax_loop, 0)

    def zero_t(j, _):
        table[j, :] = jnp.zeros((16,), jnp.float32)
        return 0
    lax.fori_loop(0, _B, zero_t, 0)

    def exploop(i, _):
        b = bbuf[pl.ds(i * 16, 16)]
        l = lbuf[pl.ds(i * 16, 16)]
        ex = jnp.exp(l - plsc.load_gather(gmax, [b]))
        plsc.addupdate_scatter(table, [b, lanes], ex)
        return 0
    lax.fori_loop(0, nr1, exploop, 0)

    def red_sum(g, _):
        rows = g * 16 + lanes
        m = jnp.zeros((16,), jnp.float32)
        for col in range(16):
            m = m + plsc.load_gather(table, [rows, jnp.full((16,), col, jnp.int32)])
        part[pl.ds(g * 16, 16)] = m
        return 0
    lax.fori_loop(0, _B // 16, red_sum, 0)

    pltpu.sync_copy(part, shsc.at[s])
    plsc.subcore_barrier()
    pltpu.sync_copy(shsc, cbuf)
    plsc.subcore_barrier()

    def invd_loop(g, _):
        m = cbuf[0, pl.ds(g * 16, 16)]
        for t in range(1, 16):
            m = m + cbuf[t, pl.ds(g * 16, 16)]
        invd[pl.ds(g * 16, 16)] = 1.0 / m
        return 0
    lax.fori_loop(0, _B // 16, invd_loop, 0)

    o3 = w * _PL_CNT
    nr3 = jnp.where(w == 31, _PL_LAST // 16, _PL_CNT // 16)

    @pl.when(w < 31)
    def _():
        pltpu.sync_copy(batch_hbm.at[pl.ds(o3, _PL_CNT)],
                        bbuf.at[pl.ds(0, _PL_CNT)])
        pltpu.sync_copy(logit_hbm.at[pl.ds(o3, _PL_CNT)],
                        lbuf.at[pl.ds(0, _PL_CNT)])

    @pl.when(w == 31)
    def _():
        pltpu.sync_copy(batch_hbm.at[pl.ds(o3, _PL_LAST)],
                        bbuf.at[pl.ds(0, _PL_LAST)])
        pltpu.sync_copy(logit_hbm.at[pl.ds(o3, _PL_LAST)],
                        lbuf.at[pl.ds(0, _PL_LAST)])

    def attn_loop(i, _):
        b = bbuf[pl.ds(i * 16, 16)]
        l = lbuf[pl.ds(i * 16, 16)]
        a = jnp.exp(l - plsc.load_gather(gmax, [b])) * plsc.load_gather(invd, [b])
        abuf[pl.ds(i * 16, 16)] = a
        return 0
    lax.fori_loop(0, nr3, attn_loop, 0)

    def zero_acc(j, _):
        for k in range(8):
            acc[j, pl.ds(k * 16, 16)] = jnp.zeros((16,), jnp.float32)
        return 0
    lax.fori_loop(0, _B, zero_acc, 0)

    def group_body(xb, i, g):
        b16 = bbuf[pl.ds(i * 16, 16)]
        a16 = abuf[pl.ds(i * 16, 16)]
        b_first = b16[0]
        b_last = b16[15]

        @pl.when(b_first == b_last)
        def _():
            regs = [jnp.zeros((16,), jnp.float32) for _ in range(8)]
            for j in range(16):
                a = a16[j]
                for k in range(8):
                    regs[k] = regs[k] + a * xb[g * 16 + j, pl.ds(k * 16, 16)]
            for k in range(8):
                plsc.addupdate(acc.at[b_first, pl.ds(k * 16, 16)], regs[k])

        @pl.when(b_first != b_last)
        def _():
            for j in range(16):
                b = b16[j]
                a = a16[j]
                for k in range(8):
                    plsc.addupdate(acc.at[b, pl.ds(k * 16, 16)],
                                   a * xb[g * 16 + j, pl.ds(k * 16, 16)])

    xbase = o3

    def start_chunk(ci, buf):
        pltpu.async_copy(x_hbm.at[pl.ds(xbase + ci * _XCH, _XCH), :], buf, xsem)

    def wait_chunk(ci, buf):
        pltpu.make_async_copy(x_hbm.at[pl.ds(xbase + ci * _XCH, _XCH), :],
                              buf, xsem).wait()

    def proc_chunk(ci, buf):
        def gb(g, _):
            group_body(buf, ci * _XG + g, g)
            return 0
        lax.fori_loop(0, _XG, gb, 0)

    nfull = jnp.where(w == 31, _PL_NCH_LAST, _PL_NCH)
    start_chunk(0, xbuf)

    def chunk_loop(ci, _):
        @pl.when(ci % 2 == 0)
        def _():
            wait_chunk(ci, xbuf)

            @pl.when(ci + 1 < nfull)
            def _():
                start_chunk(ci + 1, xbuf2)
            proc_chunk(ci, xbuf)

        @pl.when(ci % 2 == 1)
        def _():
            wait_chunk(ci, xbuf2)

            @pl.when(ci + 1 < nfull)
            def _():
                start_chunk(ci + 1, xbuf)
            proc_chunk(ci, xbuf2)
        return 0
    lax.fori_loop(0, nfull, chunk_loop, 0)

    @pl.when(w == 31)
    def _():
        pltpu.sync_copy(x_hbm.at[pl.ds(xbase + _PL_NCH_LAST * _XCH, _PL_TAIL), :],
                        xbuf.at[pl.ds(0, _PL_TAIL), :])

        def gb(g, _):
            group_body(xbuf, _PL_NCH_LAST * _XG + g, g)
            return 0
        lax.fori_loop(0, _PL_TAIL // 16, gb, 0)

    pltpu.sync_copy(acc, part_hbm.at[w])


_sc_pool = pl.kernel(
    _sc_body,
    out_type=jax.ShapeDtypeStruct((32, _B, _D), jnp.float32),
    mesh=plsc.VectorSubcoreMesh(core_axis_name="c", subcore_axis_name="s",
                                num_cores=2, num_subcores=16),
    compiler_params=pltpu.CompilerParams(needs_layout_passes=False),
    scratch_types=[
        pltpu.VMEM((_ST_CNT,), jnp.int32),
        pltpu.VMEM((_ST_CNT,), jnp.float32),
        pltpu.VMEM((_PL_CNT,), jnp.float32),
        pltpu.VMEM((_B, 16), jnp.float32),
        pltpu.VMEM((16, _B), jnp.float32),
        pltpu.VMEM((_B,), jnp.float32),
        pltpu.VMEM((_B,), jnp.float32),
        pltpu.VMEM((_B,), jnp.float32),
        pltpu.VMEM((_B, _D), jnp.float32),
        pltpu.VMEM((_XCH, _D), jnp.float32),
        pltpu.VMEM((_XCH, _D), jnp.float32),
        pltpu.SemaphoreType.DMA,
        pltpu.VMEM_SHARED((16, _B), jnp.float32),
    ],
)


def kernel(x, batch, w_x, bias):
    n, d = x.shape
    batch2 = batch.astype(jnp.int32)
    w2 = w_x.reshape(d, 1)
    b2 = bias.reshape(1, 1)

    logits = pl.pallas_call(
        _k1_logits,
        grid=(n // _BLK,),
        in_specs=[pl.BlockSpec((_BLK, d), lambda i: (i, 0)),
                  pl.BlockSpec((d, 1), lambda i: (0, 0)),
                  pl.BlockSpec((1, 1), lambda i: (0, 0))],
        out_specs=pl.BlockSpec((_BLK, 1), lambda i: (i, 0)),
        out_shape=jax.ShapeDtypeStruct((n, 1), jnp.float32),
    )(x, w2, b2)

    partials = _sc_pool(x, batch2, logits.reshape(n))

    out = pl.pallas_call(
        _k2_sum,
        in_specs=[pl.BlockSpec((32, _B, d), lambda: (0, 0, 0))],
        out_specs=pl.BlockSpec((_B, d), lambda: (0, 0)),
        out_shape=jax.ShapeDtypeStruct((_B, d), jnp.float32),
    )(partials)

    return out
